# baseline (speedup 1.0000x reference)
import functools

import jax
import jax.numpy as jnp
from jax import lax
from jax.experimental import pallas as pl
from jax.experimental.pallas import tpu as pltpu
from jax.experimental.pallas import tpu_sc as plsc

_D = 256
_K = 8192
_TOKENS = 8192
_TBLK = 128
_NW = 32
_BPW = _TOKENS // _NW
_IDX_CHUNK = 128


_KC = 128
_DCH = 2048


def _argmin_body(z_ref, w_ref, z2h_ref, w2h_ref, idx_ref, loss_ref,
                 wtb_ref, acc_ref):
    i = pl.program_id(0)

    @pl.when(i == 0)
    def _():
        wtb_ref[...] = w_ref[...].T

    zb = z_ref[...]
    z2h = z2h_ref[...]
    w2h = w2h_ref[...]
    run_v = jnp.full((_TBLK, _KC), jnp.inf, jnp.float32)
    run_c = jnp.zeros((_TBLK, _KC), jnp.int32)
    for dc in range(_K // _DCH):
        mm = jnp.dot(zb, wtb_ref[:, dc * _DCH:(dc + 1) * _DCH],
                     preferred_element_type=jnp.float32)
        for s in range(_DCH // _KC):
            c = dc * (_DCH // _KC) + s
            d = ((z2h - mm[:, s * _KC:(s + 1) * _KC])
                 + w2h[c * _KC:(c + 1) * _KC][None, :])
            lt = d < run_v
            run_v = jnp.where(lt, d, run_v)
            run_c = jnp.where(lt, jnp.int32(c), run_c)
    minv = jnp.min(run_v, axis=1, keepdims=True)
    lane = lax.broadcasted_iota(jnp.int32, (_TBLK, _KC), 1)
    gidx = run_c * _KC + lane
    idx_ref[...] = jnp.min(jnp.where(run_v == minv, gidx, jnp.int32(_K)),
                           axis=1)

    s = jnp.sum(minv)

    @pl.when(i == 0)
    def _():
        acc_ref[0, 0] = s

    @pl.when(i > 0)
    def _():
        acc_ref[0, 0] = acc_ref[0, 0] + s

    @pl.when(i == pl.num_programs(0) - 1)
    def _():
        e = acc_ref[0, 0] * (2.0 / float(_TOKENS * _D))
        loss_ref[0, 0] = e + 0.25 * e


def _argmin_call(zf, W, z2h, w2h):
    return pl.pallas_call(
        _argmin_body,
        grid=(_TOKENS // _TBLK,),
        in_specs=[
            pl.BlockSpec((_TBLK, _D), lambda i: (i, 0)),
            pl.BlockSpec((_K, _D), lambda i: (0, 0)),
            pl.BlockSpec((_TBLK, 1), lambda i: (i, 0)),
            pl.BlockSpec((_K,), lambda i: (0,)),
        ],
        out_specs=[
            pl.BlockSpec((_TBLK,), lambda i: (i,)),
            pl.BlockSpec(memory_space=pltpu.SMEM),
        ],
        out_shape=[
            jax.ShapeDtypeStruct((_TOKENS,), jnp.int32),
            jax.ShapeDtypeStruct((1, 1), jnp.float32),
        ],
        scratch_shapes=[
            pltpu.VMEM((_D, _K), jnp.float32),
            pltpu.SMEM((1, 1), jnp.float32),
        ],
    )(zf, W, z2h, w2h)


def _sc_gather_body(w_hbm, idx_hbm, out_hbm, idx_v, rows_v, sem):
    wid = lax.axis_index("s") * 2 + lax.axis_index("c")
    pltpu.sync_copy(idx_hbm.at[wid], idx_v)
    copies = [
        pltpu.async_copy(w_hbm.at[idx_v.at[j]],
                         rows_v.at[pl.ds(j * _IDX_CHUNK, _IDX_CHUNK)], sem)
        for j in range(_BPW // _IDX_CHUNK)
    ]
    for cp in copies:
        cp.wait()
    pltpu.sync_copy(rows_v, out_hbm.at[pl.ds(wid * _BPW, _BPW)])


@functools.cache
def _sc_gather_kernel():
    mesh = plsc.VectorSubcoreMesh(core_axis_name="c", subcore_axis_name="s")
    return pl.kernel(
        _sc_gather_body,
        mesh=mesh,
        out_type=jax.ShapeDtypeStruct((_TOKENS, _D), jnp.float32),
        scratch_types=[
            pltpu.VMEM((_BPW // _IDX_CHUNK, _IDX_CHUNK), jnp.int32),
            pltpu.VMEM((_BPW, _D), jnp.float32),
            pltpu.SemaphoreType.DMA,
        ],
    )


def kernel(z, W):
    zf = z.reshape(_TOKENS, _D)
    z2h = 0.5 * jnp.sum(zf ** 2, axis=1, keepdims=True)
    w2h = 0.5 * jnp.sum(W ** 2, axis=1)
    idx, loss = _argmin_call(zf, W, z2h, w2h)
    q = _sc_gather_kernel()(W, idx.reshape(_NW, _BPW // _IDX_CHUNK, _IDX_CHUNK))
    return q.reshape(z.shape), loss.reshape(())


# speedup vs baseline: 1.2560x; 1.2560x over previous
import functools

import jax
import jax.numpy as jnp
from jax import lax
from jax.experimental import pallas as pl
from jax.experimental.pallas import tpu as pltpu
from jax.experimental.pallas import tpu_sc as plsc

_D = 256
_K = 8192
_TOKENS = 8192
_TBLK = 512
_NW = 32
_BPW = _TOKENS // _NW
_IDX_CHUNK = 128


_KC = 128
_DCH = 2048


def _argmin_body(z_ref, w_ref, z2h_ref, w2h_ref, idx_ref, loss_ref,
                 wtb_ref, acc_ref):
    i = pl.program_id(0)

    @pl.when(i == 0)
    def _():
        wtb_ref[...] = w_ref[...].T

    zb = z_ref[...]
    z2h = z2h_ref[...]
    w2h = w2h_ref[...]
    run_v = jnp.full((_TBLK, _KC), jnp.inf, jnp.float32)
    run_c = jnp.zeros((_TBLK, _KC), jnp.int32)
    for dc in range(_K // _DCH):
        mm = jnp.dot(zb, wtb_ref[:, dc * _DCH:(dc + 1) * _DCH],
                     preferred_element_type=jnp.float32)
        for s in range(_DCH // _KC):
            c = dc * (_DCH // _KC) + s
            d = ((z2h - mm[:, s * _KC:(s + 1) * _KC])
                 + w2h[c * _KC:(c + 1) * _KC][None, :])
            lt = d < run_v
            run_v = jnp.where(lt, d, run_v)
            run_c = jnp.where(lt, jnp.int32(c), run_c)
    minv = jnp.min(run_v, axis=1, keepdims=True)
    lane = lax.broadcasted_iota(jnp.int32, (_TBLK, _KC), 1)
    gidx = run_c * _KC + lane
    idx_ref[...] = jnp.min(jnp.where(run_v == minv, gidx, jnp.int32(_K)),
                           axis=1)

    s = jnp.sum(minv)

    @pl.when(i == 0)
    def _():
        acc_ref[0, 0] = s

    @pl.when(i > 0)
    def _():
        acc_ref[0, 0] = acc_ref[0, 0] + s

    @pl.when(i == pl.num_programs(0) - 1)
    def _():
        e = acc_ref[0, 0] * (2.0 / float(_TOKENS * _D))
        loss_ref[0, 0] = e + 0.25 * e


def _argmin_call(zf, W, z2h, w2h):
    return pl.pallas_call(
        _argmin_body,
        grid=(_TOKENS // _TBLK,),
        in_specs=[
            pl.BlockSpec((_TBLK, _D), lambda i: (i, 0)),
            pl.BlockSpec((_K, _D), lambda i: (0, 0)),
            pl.BlockSpec((_TBLK, 1), lambda i: (i, 0)),
            pl.BlockSpec((_K,), lambda i: (0,)),
        ],
        out_specs=[
            pl.BlockSpec((_TBLK,), lambda i: (i,)),
            pl.BlockSpec(memory_space=pltpu.SMEM),
        ],
        out_shape=[
            jax.ShapeDtypeStruct((_TOKENS,), jnp.int32),
            jax.ShapeDtypeStruct((1, 1), jnp.float32),
        ],
        scratch_shapes=[
            pltpu.VMEM((_D, _K), jnp.float32),
            pltpu.SMEM((1, 1), jnp.float32),
        ],
    )(zf, W, z2h, w2h)


def _sc_gather_body(w_hbm, idx_hbm, out_hbm, idx_v, rows_v, sem):
    wid = lax.axis_index("s") * 2 + lax.axis_index("c")
    pltpu.sync_copy(idx_hbm.at[wid], idx_v)
    copies = [
        pltpu.async_copy(w_hbm.at[idx_v.at[j]],
                         rows_v.at[pl.ds(j * _IDX_CHUNK, _IDX_CHUNK)], sem)
        for j in range(_BPW // _IDX_CHUNK)
    ]
    for cp in copies:
        cp.wait()
    pltpu.sync_copy(rows_v, out_hbm.at[pl.ds(wid * _BPW, _BPW)])


@functools.cache
def _sc_gather_kernel():
    mesh = plsc.VectorSubcoreMesh(core_axis_name="c", subcore_axis_name="s")
    return pl.kernel(
        _sc_gather_body,
        mesh=mesh,
        out_type=jax.ShapeDtypeStruct((_TOKENS, _D), jnp.float32),
        scratch_types=[
            pltpu.VMEM((_BPW // _IDX_CHUNK, _IDX_CHUNK), jnp.int32),
            pltpu.VMEM((_BPW, _D), jnp.float32),
            pltpu.SemaphoreType.DMA,
        ],
    )


def kernel(z, W):
    zf = z.reshape(_TOKENS, _D)
    z2h = 0.5 * jnp.sum(zf ** 2, axis=1, keepdims=True)
    w2h = 0.5 * jnp.sum(W ** 2, axis=1)
    idx, loss = _argmin_call(zf, W, z2h, w2h)
    q = _sc_gather_kernel()(W, idx.reshape(_NW, _BPW // _IDX_CHUNK, _IDX_CHUNK))
    return q.reshape(z.shape), loss.reshape(())
